# Initial kernel scaffold; baseline (speedup 1.0000x reference)
import jax
import jax.numpy as jnp
from jax.experimental import pallas as pl


def kernel(x, adj, We1, be1, We2, be2, We3, be3, Wz, bz, Wd1, bd1, Wd2, bd2, Wd3, bd3, Wxb, bxb, Wg1, Wg2, Wg3, Wg4, Wg5, Wg6, Wg7, Wg8, cluster):
    raise NotImplementedError("write your pallas kernel here")


# speedup vs baseline: 1.2074x; 1.2074x over previous
import functools

import jax
import jax.numpy as jnp
from jax.experimental import pallas as pl

N = 10000
A = 0.5
V = 1.0

BM = 400
BM_CAST = 200
BM_Q = 2000

_bf = jnp.bfloat16
_f32 = jnp.float32


def _dot(a, b):
    return jnp.dot(a, b, preferred_element_type=_f32)


def _ae_body(x_ref,
             we1, be1, we2, be2, we3, be3, wz, bz,
             wd1, bd1, wd2, bd2, wd3, bd3, wxb, bxb, wg1,
             h1_ref, h2_ref, h3_ref, zae_ref, xbar_ref, t1_ref):
    xb = x_ref[...].astype(_bf)
    h1 = jax.nn.relu(_dot(xb, we1[...]) + be1[...])
    h1_ref[...] = h1.astype(_bf)
    h2 = jax.nn.relu(_dot(h1.astype(_bf), we2[...]) + be2[...])
    h2_ref[...] = h2.astype(_bf)
    h3 = jax.nn.relu(_dot(h2.astype(_bf), we3[...]) + be3[...])
    h3_ref[...] = h3.astype(_bf)
    zae = _dot(h3.astype(_bf), wz[...]) + bz[...]
    zae_ref[...] = zae
    dd1 = jax.nn.relu(_dot(zae.astype(_bf), wd1[...]) + bd1[...])
    dd2 = jax.nn.relu(_dot(dd1.astype(_bf), wd2[...]) + bd2[...])
    dd3 = jax.nn.relu(_dot(dd2.astype(_bf), wd3[...]) + bd3[...])
    xbar_ref[...] = _dot(dd3.astype(_bf), wxb[...]) + bxb[...]
    t1_ref[...] = _dot(xb, wg1[...]).astype(_bf)


def _run_ae(x, wb, bb, wg1b):
    g = N // BM
    full = lambda arr: pl.BlockSpec(arr.shape, lambda i: (0,) * arr.ndim)
    row = lambda c, dt=None: pl.BlockSpec((BM, c), lambda i: (i, 0))
    in_specs = [row(512)]
    args = [x]
    for w, b in zip(wb, bb):
        in_specs += [full(w), full(b)]
        args += [w, b]
    in_specs.append(full(wg1b))
    args.append(wg1b)
    out_shape = [
        jax.ShapeDtypeStruct((N, 128), _bf),
        jax.ShapeDtypeStruct((N, 256), _bf),
        jax.ShapeDtypeStruct((N, 512), _bf),
        jax.ShapeDtypeStruct((N, 64), _f32),
        jax.ShapeDtypeStruct((N, 512), _f32),
        jax.ShapeDtypeStruct((N, 128), _bf),
    ]
    out_specs = [row(128), row(256), row(512), row(64), row(512), row(128)]
    return pl.pallas_call(
        _ae_body, grid=(g,), in_specs=in_specs, out_specs=out_specs,
        out_shape=out_shape)(*args)


def _cast_spmm1_body(adj_ref, t1_ref, h1_ref, wg2_ref, adjb_ref, t2_ref):
    ab = adj_ref[...].astype(_bf)
    adjb_ref[...] = ab
    ge1 = jax.nn.relu(_dot(ab, t1_ref[...]))
    mix = ((1.0 - A) * ge1 + A * h1_ref[...].astype(_f32)).astype(_bf)
    t2_ref[...] = _dot(mix, wg2_ref[...]).astype(_bf)


def _run_cast_spmm1(adj, t1b, h1b, wg2b):
    g = N // BM_CAST
    return pl.pallas_call(
        _cast_spmm1_body, grid=(g,),
        in_specs=[
            pl.BlockSpec((BM_CAST, N), lambda i: (i, 0)),
            pl.BlockSpec((N, 128), lambda i: (0, 0)),
            pl.BlockSpec((BM_CAST, 128), lambda i: (i, 0)),
            pl.BlockSpec((128, 256), lambda i: (0, 0)),
        ],
        out_specs=[
            pl.BlockSpec((BM_CAST, N), lambda i: (i, 0)),
            pl.BlockSpec((BM_CAST, 256), lambda i: (i, 0)),
        ],
        out_shape=[
            jax.ShapeDtypeStruct((N, N), _bf),
            jax.ShapeDtypeStruct((N, 256), _bf),
        ])(adj, t1b, h1b, wg2b)


def _spmm_mix_body(adj_ref, t_ref, h_ref, w_ref, out_ref):
    acc = jax.nn.relu(_dot(adj_ref[...], t_ref[...]))
    mix = ((1.0 - A) * acc + A * h_ref[...].astype(_f32)).astype(_bf)
    out_ref[...] = _dot(mix, w_ref[...]).astype(_bf)


def _spmm_mat_body(adj_ref, t_ref, w_ref, out_ref):
    acc = jax.nn.relu(_dot(adj_ref[...], t_ref[...]))
    out_ref[...] = _dot(acc.astype(_bf), w_ref[...]).astype(_bf)


def _spmm_relu_body(adj_ref, t_ref, out_ref):
    out_ref[...] = jax.nn.relu(_dot(adj_ref[...], t_ref[...]))


def _run_spmm(body, adjb, t, extras, out_cols, out_dtype):
    g = N // BM
    in_specs = [pl.BlockSpec((BM, N), lambda i: (i, 0)),
                pl.BlockSpec(t.shape, lambda i: (0, 0))]
    args = [adjb, t]
    for e, blocked in extras:
        if blocked:
            in_specs.append(pl.BlockSpec((BM, e.shape[1]), lambda i: (i, 0)))
        else:
            in_specs.append(pl.BlockSpec(e.shape, lambda i: (0, 0)))
        args.append(e)
    return pl.pallas_call(
        body, grid=(g,), in_specs=in_specs,
        out_specs=pl.BlockSpec((BM, out_cols), lambda i: (i, 0)),
        out_shape=jax.ShapeDtypeStruct((N, out_cols), out_dtype))(*args)


def _spmm_zgae_body(adj_ref, t_ref, zae_ref, wg5_ref,
                    zgb_ref, zi_ref, t5_ref):
    zg = _dot(adj_ref[...], t_ref[...])
    zgb = zg.astype(_bf)
    zgb_ref[...] = zgb
    zi_ref[...] = ((1.0 - A) * zg + A * zae_ref[...]).astype(_bf)
    t5_ref[...] = _dot(zgb, wg5_ref[...]).astype(_bf)


def _run_spmm_zgae(adjb, t4b, zae, wg5b):
    g = N // BM
    return pl.pallas_call(
        _spmm_zgae_body, grid=(g,),
        in_specs=[
            pl.BlockSpec((BM, N), lambda i: (i, 0)),
            pl.BlockSpec((N, 64), lambda i: (0, 0)),
            pl.BlockSpec((BM, 64), lambda i: (i, 0)),
            pl.BlockSpec((64, 512), lambda i: (0, 0)),
        ],
        out_specs=[
            pl.BlockSpec((BM, 64), lambda i: (i, 0)),
            pl.BlockSpec((BM, 64), lambda i: (i, 0)),
            pl.BlockSpec((BM, 512), lambda i: (i, 0)),
        ],
        out_shape=[
            jax.ShapeDtypeStruct((N, 64), _bf),
            jax.ShapeDtypeStruct((N, 64), _bf),
            jax.ShapeDtypeStruct((N, 512), _bf),
        ])(adjb, t4b, zae, wg5b)


def _spmm_zl_gd1_body(adj_ref, zi_ref, t5_ref, wg6_ref, zl_ref, t6_ref):
    ab = adj_ref[...]
    zl_ref[...] = _dot(ab, zi_ref[...])
    gd1 = jax.nn.relu(_dot(ab, t5_ref[...])).astype(_bf)
    t6_ref[...] = _dot(gd1, wg6_ref[...]).astype(_bf)


def _run_spmm_zl_gd1(adjb, zib, t5b, wg6b):
    g = N // BM
    return pl.pallas_call(
        _spmm_zl_gd1_body, grid=(g,),
        in_specs=[
            pl.BlockSpec((BM, N), lambda i: (i, 0)),
            pl.BlockSpec((N, 64), lambda i: (0, 0)),
            pl.BlockSpec((N, 512), lambda i: (0, 0)),
            pl.BlockSpec((512, 256), lambda i: (0, 0)),
        ],
        out_specs=[
            pl.BlockSpec((BM, 64), lambda i: (i, 0)),
            pl.BlockSpec((BM, 256), lambda i: (i, 0)),
        ],
        out_shape=[
            jax.ShapeDtypeStruct((N, 64), _f32),
            jax.ShapeDtypeStruct((N, 256), _bf),
        ])(adjb, zib, t5b, wg6b)


def _adj_hat_body(zb_ref, zfull_ref, out_ref):
    prod = jax.lax.dot_general(
        zb_ref[...], zfull_ref[...], (((1,), (1,)), ((), ())),
        preferred_element_type=_f32)
    out_ref[...] = jax.nn.sigmoid(prod)


def _run_adj_hat(zgb):
    g = N // BM
    return pl.pallas_call(
        _adj_hat_body, grid=(g,),
        in_specs=[
            pl.BlockSpec((BM, 64), lambda i: (i, 0)),
            pl.BlockSpec((N, 64), lambda i: (0, 0)),
        ],
        out_specs=pl.BlockSpec((BM, N), lambda i: (i, 0)),
        out_shape=jax.ShapeDtypeStruct((N, N), _f32))(zgb, zgb)


def _softq_body(z_ref, c_ref, q_ref):
    z = z_ref[...]
    c = c_ref[...]
    zz = jnp.sum(z * z, axis=1, keepdims=True)
    cc = jnp.sum(c * c, axis=1)[None, :]
    zc = jax.lax.dot_general(z, c, (((1,), (1,)), ((), ())),
                             preferred_element_type=_f32,
                             precision=jax.lax.Precision.HIGHEST)
    d2 = zz + cc - 2.0 * zc
    q = 1.0 / (1.0 + d2 / V)
    q = q ** ((V + 1.0) / 2.0)
    q_ref[...] = q / jnp.sum(q, axis=1, keepdims=True)


def _run_softq(z, cluster):
    g = N // BM_Q
    return pl.pallas_call(
        _softq_body, grid=(g,),
        in_specs=[
            pl.BlockSpec((BM_Q, 64), lambda i: (i, 0)),
            pl.BlockSpec((16, 64), lambda i: (0, 0)),
        ],
        out_specs=pl.BlockSpec((BM_Q, 16), lambda i: (i, 0)),
        out_shape=jax.ShapeDtypeStruct((N, 16), _f32))(z, cluster)


def kernel(x, adj, We1, be1, We2, be2, We3, be3, Wz, bz, Wd1, bd1, Wd2, bd2,
           Wd3, bd3, Wxb, bxb, Wg1, Wg2, Wg3, Wg4, Wg5, Wg6, Wg7, Wg8,
           cluster):
    wb = [w.astype(_bf) for w in (We1, We2, We3, Wz, Wd1, Wd2, Wd3, Wxb)]
    bb = [b.reshape(1, -1) for b in (be1, be2, be3, bz, bd1, bd2, bd3, bxb)]
    wg = [w.astype(_bf) for w in (Wg1, Wg2, Wg3, Wg4, Wg5, Wg6, Wg7, Wg8)]

    h1b, h2b, h3b, z_ae, x_bar, t1b = _run_ae(x, wb, bb, wg[0])

    adjb, t2b = _run_cast_spmm1(adj, t1b, h1b, wg[1])
    t3b = _run_spmm(_spmm_mix_body, adjb, t2b,
                    [(h2b, True), (wg[2], False)], 512, _bf)
    t4b = _run_spmm(_spmm_mix_body, adjb, t3b,
                    [(h3b, True), (wg[3], False)], 64, _bf)
    zgb, zib, t5b = _run_spmm_zgae(adjb, t4b, z_ae, wg[4])
    z_l, t6b = _run_spmm_zl_gd1(adjb, zib, t5b, wg[5])
    t7b = _run_spmm(_spmm_mat_body, adjb, t6b, [(wg[6], False)], 128, _bf)
    t8b = _run_spmm(_spmm_mat_body, adjb, t7b, [(wg[7], False)], 512, _bf)
    z_hat = _run_spmm(_spmm_relu_body, adjb, t8b, [], 512, _f32)

    adj_hat = _run_adj_hat(zgb)
    q = _run_softq(z_l, cluster)
    q1 = _run_softq(z_ae, cluster)
    return (x_bar, z_hat, adj_hat, z_ae, q, q1, z_l)


# speedup vs baseline: 1.4253x; 1.1804x over previous
import jax
import jax.numpy as jnp
from jax.experimental import pallas as pl

N = 10000
A = 0.5
V = 1.0

BM = 400
BM_CAST = 200
BM_Q = 2000

_bf = jnp.bfloat16
_f32 = jnp.float32


def _dot(a, b):
    return jnp.dot(a, b, preferred_element_type=_f32)


def _row(c):
    return pl.BlockSpec((BM, c), lambda i: (i, 0))


def _full(arr):
    return pl.BlockSpec(arr.shape, lambda i: (0,) * arr.ndim)


def _ae_body(x_ref,
             we1, be1, we2, be2, we3, be3, wz, bz,
             wd1, bd1, wd2, bd2, wd3, bd3, wxb, bxb, wg1,
             h1_ref, h2_ref, h3_ref, zae_ref, xbar_ref, m1_ref):
    xb = x_ref[...].astype(_bf)
    h1 = jax.nn.relu(_dot(xb, we1[...]) + be1[...])
    h1_ref[...] = h1.astype(_bf)
    h2 = jax.nn.relu(_dot(h1.astype(_bf), we2[...]) + be2[...])
    h2_ref[...] = h2.astype(_bf)
    h3 = jax.nn.relu(_dot(h2.astype(_bf), we3[...]) + be3[...])
    h3_ref[...] = h3.astype(_bf)
    zae = _dot(h3.astype(_bf), wz[...]) + bz[...]
    zae_ref[...] = zae
    dd1 = jax.nn.relu(_dot(zae.astype(_bf), wd1[...]) + bd1[...])
    dd2 = jax.nn.relu(_dot(dd1.astype(_bf), wd2[...]) + bd2[...])
    dd3 = jax.nn.relu(_dot(dd2.astype(_bf), wd3[...]) + bd3[...])
    xbar_ref[...] = _dot(dd3.astype(_bf), wxb[...]) + bxb[...]
    m1_ref[...] = _dot(xb, wg1[...]).astype(_bf)


def _run_ae(x, wb, bb, wg1b):
    in_specs = [_row(512)]
    args = [x]
    for w, b in zip(wb, bb):
        in_specs += [_full(w), _full(b)]
        args += [w, b]
    in_specs.append(_full(wg1b))
    args.append(wg1b)
    out_shape = [
        jax.ShapeDtypeStruct((N, 128), _bf),
        jax.ShapeDtypeStruct((N, 256), _bf),
        jax.ShapeDtypeStruct((N, 512), _bf),
        jax.ShapeDtypeStruct((N, 64), _f32),
        jax.ShapeDtypeStruct((N, 512), _f32),
        jax.ShapeDtypeStruct((N, 128), _bf),
    ]
    out_specs = [_row(128), _row(256), _row(512), _row(64), _row(512),
                 _row(128)]
    return pl.pallas_call(
        _ae_body, grid=(N // BM,), in_specs=in_specs, out_specs=out_specs,
        out_shape=out_shape)(*args)


def _cast_l1_body(adj_ref, m1_ref, h1_ref, adjb_ref, m2_ref):
    ab = adj_ref[...].astype(_bf)
    adjb_ref[...] = ab
    ge1 = jax.nn.relu(_dot(ab, m1_ref[...]))
    m2_ref[...] = ((1.0 - A) * ge1
                   + A * h1_ref[...].astype(_f32)).astype(_bf)


def _run_cast_l1(adj, m1b, h1b):
    return pl.pallas_call(
        _cast_l1_body, grid=(N // BM_CAST,),
        in_specs=[
            pl.BlockSpec((BM_CAST, N), lambda i: (i, 0)),
            pl.BlockSpec((N, 128), lambda i: (0, 0)),
            pl.BlockSpec((BM_CAST, 128), lambda i: (i, 0)),
        ],
        out_specs=[
            pl.BlockSpec((BM_CAST, N), lambda i: (i, 0)),
            pl.BlockSpec((BM_CAST, 128), lambda i: (i, 0)),
        ],
        out_shape=[
            jax.ShapeDtypeStruct((N, N), _bf),
            jax.ShapeDtypeStruct((N, 128), _bf),
        ])(adj, m1b, h1b)


def _l2_body(adj_ref, m_ref, w_ref, h_ref, out_ref):
    acc = _dot(adj_ref[...], m_ref[...]).astype(_bf)
    ge = jax.nn.relu(_dot(acc, w_ref[...]))
    out_ref[...] = ((1.0 - A) * ge + A * h_ref[...].astype(_f32)).astype(_bf)


def _l3_body(adj_ref, m_ref, w_ref, h_ref, w2_ref, out_ref):
    acc = _dot(adj_ref[...], m_ref[...]).astype(_bf)
    ge = jax.nn.relu(_dot(acc, w_ref[...]))
    mix = ((1.0 - A) * ge + A * h_ref[...].astype(_f32)).astype(_bf)
    out_ref[...] = _dot(mix, w2_ref[...]).astype(_bf)


def _l4_body(adj_ref, m_ref, zae_ref, zgb_ref, m5_ref):
    zg = _dot(adj_ref[...], m_ref[...])
    zgb = zg.astype(_bf)
    zgb_ref[...] = zgb
    zi = ((1.0 - A) * zg + A * zae_ref[...]).astype(_bf)
    m5_ref[...] = jnp.concatenate([zi, zgb], axis=1)


def _l5_body(adj_ref, m_ref, w5_ref, w6_ref, zl_ref, m6_ref):
    acc = _dot(adj_ref[...], m_ref[...])
    zl_ref[...] = acc[:, :64]
    gd1 = jax.nn.relu(_dot(acc[:, 64:].astype(_bf), w5_ref[...])).astype(_bf)
    m6_ref[...] = _dot(gd1, w6_ref[...]).astype(_bf)


def _l6_body(adj_ref, m_ref, w_ref, out_ref):
    gd2 = jax.nn.relu(_dot(adj_ref[...], m_ref[...])).astype(_bf)
    out_ref[...] = _dot(gd2, w_ref[...]).astype(_bf)


def _l7_body(adj_ref, m_ref, out_ref):
    out_ref[...] = jax.nn.relu(_dot(adj_ref[...], m_ref[...])).astype(_bf)


def _l8_body(adj_ref, m_ref, w_ref, out_ref):
    acc = _dot(adj_ref[...], m_ref[...]).astype(_bf)
    out_ref[...] = jax.nn.relu(_dot(acc, w_ref[...]))


def _spmm(body, adjb, m, extras, outs):
    in_specs = [_row(N), _full(m)]
    args = [adjb, m]
    for e, blocked in extras:
        in_specs.append(_row(e.shape[1]) if blocked else _full(e))
        args.append(e)
    out_specs = [_row(c) for c, _ in outs]
    out_shape = [jax.ShapeDtypeStruct((N, c), dt) for c, dt in outs]
    if len(outs) == 1:
        out_specs, out_shape = out_specs[0], out_shape[0]
    return pl.pallas_call(
        body, grid=(N // BM,), in_specs=in_specs, out_specs=out_specs,
        out_shape=out_shape)(*args)


def _adj_hat_body(zb_ref, zfull_ref, out_ref):
    prod = jax.lax.dot_general(
        zb_ref[...], zfull_ref[...], (((1,), (1,)), ((), ())),
        preferred_element_type=_f32)
    out_ref[...] = jax.nn.sigmoid(prod)


def _run_adj_hat(zgb):
    return pl.pallas_call(
        _adj_hat_body, grid=(N // BM,),
        in_specs=[_row(64), _full(zgb)],
        out_specs=_row(N),
        out_shape=jax.ShapeDtypeStruct((N, N), _f32))(zgb, zgb)


def _softq_body(z_ref, c_ref, q_ref):
    z = z_ref[...]
    c = c_ref[...]
    zz = jnp.sum(z * z, axis=1, keepdims=True)
    cc = jnp.sum(c * c, axis=1)[None, :]
    zc = jax.lax.dot_general(z, c, (((1,), (1,)), ((), ())),
                             preferred_element_type=_f32,
                             precision=jax.lax.Precision.HIGHEST)
    d2 = zz + cc - 2.0 * zc
    q = 1.0 / (1.0 + d2 / V)
    q = q ** ((V + 1.0) / 2.0)
    q_ref[...] = q / jnp.sum(q, axis=1, keepdims=True)


def _run_softq(z, cluster):
    return pl.pallas_call(
        _softq_body, grid=(N // BM_Q,),
        in_specs=[
            pl.BlockSpec((BM_Q, 64), lambda i: (i, 0)),
            pl.BlockSpec((16, 64), lambda i: (0, 0)),
        ],
        out_specs=pl.BlockSpec((BM_Q, 16), lambda i: (i, 0)),
        out_shape=jax.ShapeDtypeStruct((N, 16), _f32))(z, cluster)


def kernel(x, adj, We1, be1, We2, be2, We3, be3, Wz, bz, Wd1, bd1, Wd2, bd2,
           Wd3, bd3, Wxb, bxb, Wg1, Wg2, Wg3, Wg4, Wg5, Wg6, Wg7, Wg8,
           cluster):
    wb = [w.astype(_bf) for w in (We1, We2, We3, Wz, Wd1, Wd2, Wd3, Wxb)]
    bb = [b.reshape(1, -1) for b in (be1, be2, be3, bz, bd1, bd2, bd3, bxb)]
    wg = [w.astype(_bf) for w in (Wg1, Wg2, Wg3, Wg4, Wg5, Wg6, Wg7, Wg8)]

    h1b, h2b, h3b, z_ae, x_bar, m1b = _run_ae(x, wb, bb, wg[0])

    adjb, m2b = _run_cast_l1(adj, m1b, h1b)
    m3b = _spmm(_l2_body, adjb, m2b, [(wg[1], False), (h2b, True)],
                [(256, _bf)])
    m4b = _spmm(_l3_body, adjb, m3b,
                [(wg[2], False), (h3b, True), (wg[3], False)], [(64, _bf)])
    zgb, m5b = _spmm(_l4_body, adjb, m4b, [(z_ae, True)],
                     [(64, _bf), (128, _bf)])
    z_l, m6b = _spmm(_l5_body, adjb, m5b, [(wg[4], False), (wg[5], False)],
                     [(64, _f32), (256, _bf)])
    m7b = _spmm(_l6_body, adjb, m6b, [(wg[6], False)], [(128, _bf)])
    m8b = _spmm(_l7_body, adjb, m7b, [], [(128, _bf)])
    z_hat = _spmm(_l8_body, adjb, m8b, [(wg[7], False)], [(512, _f32)])

    adj_hat = _run_adj_hat(zgb)
    q = _run_softq(z_l, cluster)
    q1 = _run_softq(z_ae, cluster)
    return (x_bar, z_hat, adj_hat, z_ae, q, q1, z_l)


# speedup vs baseline: 1.4665x; 1.0289x over previous
import jax
import jax.numpy as jnp
from jax.experimental import pallas as pl

N = 10000
A = 0.5
V = 1.0

BM = 400
BM_CAST = 200
BM_Q = 2000

_bf = jnp.bfloat16
_f32 = jnp.float32


def _dot(a, b):
    return jnp.dot(a, b, preferred_element_type=_f32)


def _row(c):
    return pl.BlockSpec((BM, c), lambda i: (i, 0))


def _full(arr):
    return pl.BlockSpec(arr.shape, lambda i: (0,) * arr.ndim)


def _ae_body(x_ref,
             we1, be1, we2, be2, we3, be3, wz, bz,
             wd1, bd1, wd2, bd2, wd3, bd3, wxb, bxb, wg1,
             h1_ref, h2_ref, h3_ref, zae_ref, xbar_ref, m1_ref):
    xb = x_ref[...].astype(_bf)
    h1 = jax.nn.relu(_dot(xb, we1[...]) + be1[...])
    h1_ref[...] = h1.astype(_bf)
    h2 = jax.nn.relu(_dot(h1.astype(_bf), we2[...]) + be2[...])
    h2_ref[...] = h2.astype(_bf)
    h3 = jax.nn.relu(_dot(h2.astype(_bf), we3[...]) + be3[...])
    h3_ref[...] = h3.astype(_bf)
    zae = _dot(h3.astype(_bf), wz[...]) + bz[...]
    zae_ref[...] = zae
    dd1 = jax.nn.relu(_dot(zae.astype(_bf), wd1[...]) + bd1[...])
    dd2 = jax.nn.relu(_dot(dd1.astype(_bf), wd2[...]) + bd2[...])
    dd3 = jax.nn.relu(_dot(dd2.astype(_bf), wd3[...]) + bd3[...])
    xbar_ref[...] = _dot(dd3.astype(_bf), wxb[...]) + bxb[...]
    m1_ref[...] = _dot(xb, wg1[...]).astype(_bf)


def _run_ae(x, wb, bb, wg1b):
    in_specs = [_row(512)]
    args = [x]
    for w, b in zip(wb, bb):
        in_specs += [_full(w), _full(b)]
        args += [w, b]
    in_specs.append(_full(wg1b))
    args.append(wg1b)
    out_shape = [
        jax.ShapeDtypeStruct((N, 128), _bf),
        jax.ShapeDtypeStruct((N, 256), _bf),
        jax.ShapeDtypeStruct((N, 512), _bf),
        jax.ShapeDtypeStruct((N, 64), _f32),
        jax.ShapeDtypeStruct((N, 512), _f32),
        jax.ShapeDtypeStruct((N, 128), _bf),
    ]
    out_specs = [_row(128), _row(256), _row(512), _row(64), _row(512),
                 _row(128)]
    return pl.pallas_call(
        _ae_body, grid=(N // BM,), in_specs=in_specs, out_specs=out_specs,
        out_shape=out_shape)(*args)


def _cast_l1_body(adj_ref, m1_ref, h1_ref, adjb_ref, m2_ref):
    ab = adj_ref[...].astype(_bf)
    adjb_ref[...] = ab
    ge1 = jax.nn.relu(_dot(ab, m1_ref[...]))
    m2_ref[...] = ((1.0 - A) * ge1
                   + A * h1_ref[...].astype(_f32)).astype(_bf)


def _run_cast_l1(adj, m1b, h1b):
    return pl.pallas_call(
        _cast_l1_body, grid=(N // BM_CAST,),
        in_specs=[
            pl.BlockSpec((BM_CAST, N), lambda i: (i, 0)),
            pl.BlockSpec((N, 128), lambda i: (0, 0)),
            pl.BlockSpec((BM_CAST, 128), lambda i: (i, 0)),
        ],
        out_specs=[
            pl.BlockSpec((BM_CAST, N), lambda i: (i, 0)),
            pl.BlockSpec((BM_CAST, 128), lambda i: (i, 0)),
        ],
        out_shape=[
            jax.ShapeDtypeStruct((N, N), _bf),
            jax.ShapeDtypeStruct((N, 128), _bf),
        ])(adj, m1b, h1b)


def _l2_body(adj_ref, m_ref, w_ref, h_ref, out_ref):
    acc = _dot(adj_ref[...], m_ref[...]).astype(_bf)
    ge = jax.nn.relu(_dot(acc, w_ref[...]))
    out_ref[...] = ((1.0 - A) * ge + A * h_ref[...].astype(_f32)).astype(_bf)


def _l3_body(adj_ref, m_ref, w_ref, h_ref, w2_ref, out_ref):
    acc = _dot(adj_ref[...], m_ref[...]).astype(_bf)
    ge = jax.nn.relu(_dot(acc, w_ref[...]))
    mix = ((1.0 - A) * ge + A * h_ref[...].astype(_f32)).astype(_bf)
    out_ref[...] = _dot(mix, w2_ref[...]).astype(_bf)


def _l4_body(adj_ref, m_ref, zae_ref, zgb_ref, m5_ref):
    zg = _dot(adj_ref[...], m_ref[...])
    zgb = zg.astype(_bf)
    zgb_ref[...] = zgb
    zi = ((1.0 - A) * zg + A * zae_ref[...]).astype(_bf)
    m5_ref[...] = jnp.concatenate([zi, zgb], axis=1)


def _l5_body(adj_ref, m_ref, w5_ref, w6_ref, zl_ref, m6_ref):
    acc = _dot(adj_ref[...], m_ref[...])
    zl_ref[...] = acc[:, :64]
    gd1 = jax.nn.relu(_dot(acc[:, 64:].astype(_bf), w5_ref[...])).astype(_bf)
    m6_ref[...] = _dot(gd1, w6_ref[...]).astype(_bf)


def _l6_body(adj_ref, m_ref, w_ref, out_ref):
    gd2 = jax.nn.relu(_dot(adj_ref[...], m_ref[...])).astype(_bf)
    out_ref[...] = _dot(gd2, w_ref[...]).astype(_bf)


def _l7_body(adj_ref, m_ref, zgb_ref, zgfull_ref, out_ref, ah_ref):
    out_ref[...] = jax.nn.relu(_dot(adj_ref[...], m_ref[...])).astype(_bf)
    prod = jax.lax.dot_general(
        zgb_ref[...], zgfull_ref[...], (((1,), (1,)), ((), ())),
        preferred_element_type=_f32)
    ah_ref[...] = jax.nn.sigmoid(prod)


def _l8_body(adj_ref, m_ref, w_ref, out_ref):
    acc = _dot(adj_ref[...], m_ref[...]).astype(_bf)
    out_ref[...] = jax.nn.relu(_dot(acc, w_ref[...]))


def _spmm(body, adjb, m, extras, outs):
    in_specs = [_row(N), _full(m)]
    args = [adjb, m]
    for e, blocked in extras:
        in_specs.append(_row(e.shape[1]) if blocked else _full(e))
        args.append(e)
    out_specs = [_row(c) for c, _ in outs]
    out_shape = [jax.ShapeDtypeStruct((N, c), dt) for c, dt in outs]
    if len(outs) == 1:
        out_specs, out_shape = out_specs[0], out_shape[0]
    return pl.pallas_call(
        body, grid=(N // BM,), in_specs=in_specs, out_specs=out_specs,
        out_shape=out_shape)(*args)


def _softq_pair_body(za_ref, zb_ref, c_ref, qa_ref, qb_ref):
    c = c_ref[...]
    cc = jnp.sum(c * c, axis=1)[None, :]

    def one(z, q_ref):
        zz = jnp.sum(z * z, axis=1, keepdims=True)
        zc = jax.lax.dot_general(z, c, (((1,), (1,)), ((), ())),
                                 preferred_element_type=_f32,
                                 precision=jax.lax.Precision.HIGHEST)
        d2 = zz + cc - 2.0 * zc
        q = 1.0 / (1.0 + d2 / V)
        q = q ** ((V + 1.0) / 2.0)
        q_ref[...] = q / jnp.sum(q, axis=1, keepdims=True)

    one(za_ref[...], qa_ref)
    one(zb_ref[...], qb_ref)


def _run_softq_pair(za, zb, cluster):
    qspec = pl.BlockSpec((BM_Q, 16), lambda i: (i, 0))
    zspec = pl.BlockSpec((BM_Q, 64), lambda i: (i, 0))
    return pl.pallas_call(
        _softq_pair_body, grid=(N // BM_Q,),
        in_specs=[zspec, zspec, pl.BlockSpec((16, 64), lambda i: (0, 0))],
        out_specs=[qspec, qspec],
        out_shape=[jax.ShapeDtypeStruct((N, 16), _f32)] * 2)(za, zb, cluster)


def kernel(x, adj, We1, be1, We2, be2, We3, be3, Wz, bz, Wd1, bd1, Wd2, bd2,
           Wd3, bd3, Wxb, bxb, Wg1, Wg2, Wg3, Wg4, Wg5, Wg6, Wg7, Wg8,
           cluster):
    wb = [w.astype(_bf) for w in (We1, We2, We3, Wz, Wd1, Wd2, Wd3, Wxb)]
    bb = [b.reshape(1, -1) for b in (be1, be2, be3, bz, bd1, bd2, bd3, bxb)]
    wg = [w.astype(_bf) for w in (Wg1, Wg2, Wg3, Wg4, Wg5, Wg6, Wg7, Wg8)]

    h1b, h2b, h3b, z_ae, x_bar, m1b = _run_ae(x, wb, bb, wg[0])

    adjb, m2b = _run_cast_l1(adj, m1b, h1b)
    m3b = _spmm(_l2_body, adjb, m2b, [(wg[1], False), (h2b, True)],
                [(256, _bf)])
    m4b = _spmm(_l3_body, adjb, m3b,
                [(wg[2], False), (h3b, True), (wg[3], False)], [(64, _bf)])
    zgb, m5b = _spmm(_l4_body, adjb, m4b, [(z_ae, True)],
                     [(64, _bf), (128, _bf)])
    z_l, m6b = _spmm(_l5_body, adjb, m5b, [(wg[4], False), (wg[5], False)],
                     [(64, _f32), (256, _bf)])
    m7b = _spmm(_l6_body, adjb, m6b, [(wg[6], False)], [(128, _bf)])
    m8b, adj_hat = _spmm(_l7_body, adjb, m7b, [(zgb, True), (zgb, False)],
                         [(128, _bf), (N, _f32)])
    z_hat = _spmm(_l8_body, adjb, m8b, [(wg[7], False)], [(512, _f32)])

    q, q1 = _run_softq_pair(z_l, z_ae, cluster)
    return (x_bar, z_hat, adj_hat, z_ae, q, q1, z_l)


# speedup vs baseline: 1.4732x; 1.0046x over previous
import jax
import jax.numpy as jnp
from jax.experimental import pallas as pl

N = 10000
A = 0.5
V = 1.0

BM = 400
BM_CAST = 400

_bf = jnp.bfloat16
_f32 = jnp.float32


def _dot(a, b):
    return jnp.dot(a, b, preferred_element_type=_f32)


def _row(c):
    return pl.BlockSpec((BM, c), lambda i: (i, 0))


def _full(arr):
    return pl.BlockSpec(arr.shape, lambda i: (0,) * arr.ndim)


def _ae_body(x_ref,
             we1, be1, we2, be2, we3, be3, wz, bz,
             wd1, bd1, wd2, bd2, wd3, bd3, wxb, bxb, wg1,
             h1_ref, h2_ref, h3_ref, zae_ref, xbar_ref, m1_ref):
    xb = x_ref[...].astype(_bf)
    h1 = jax.nn.relu(_dot(xb, we1[...]) + be1[...])
    h1_ref[...] = h1.astype(_bf)
    h2 = jax.nn.relu(_dot(h1.astype(_bf), we2[...]) + be2[...])
    h2_ref[...] = h2.astype(_bf)
    h3 = jax.nn.relu(_dot(h2.astype(_bf), we3[...]) + be3[...])
    h3_ref[...] = h3.astype(_bf)
    zae = _dot(h3.astype(_bf), wz[...]) + bz[...]
    zae_ref[...] = zae
    dd1 = jax.nn.relu(_dot(zae.astype(_bf), wd1[...]) + bd1[...])
    dd2 = jax.nn.relu(_dot(dd1.astype(_bf), wd2[...]) + bd2[...])
    dd3 = jax.nn.relu(_dot(dd2.astype(_bf), wd3[...]) + bd3[...])
    xbar_ref[...] = _dot(dd3.astype(_bf), wxb[...]) + bxb[...]
    m1_ref[...] = _dot(xb, wg1[...]).astype(_bf)


def _run_ae(x, wb, bb, wg1b):
    in_specs = [_row(512)]
    args = [x]
    for w, b in zip(wb, bb):
        in_specs += [_full(w), _full(b)]
        args += [w, b]
    in_specs.append(_full(wg1b))
    args.append(wg1b)
    out_shape = [
        jax.ShapeDtypeStruct((N, 128), _bf),
        jax.ShapeDtypeStruct((N, 256), _bf),
        jax.ShapeDtypeStruct((N, 512), _bf),
        jax.ShapeDtypeStruct((N, 64), _f32),
        jax.ShapeDtypeStruct((N, 512), _f32),
        jax.ShapeDtypeStruct((N, 128), _bf),
    ]
    out_specs = [_row(128), _row(256), _row(512), _row(64), _row(512),
                 _row(128)]
    return pl.pallas_call(
        _ae_body, grid=(N // BM,), in_specs=in_specs, out_specs=out_specs,
        out_shape=out_shape)(*args)


def _cast_l1_body(adj_ref, m1_ref, h1_ref, adjb_ref, m2_ref):
    ab = adj_ref[...].astype(_bf)
    adjb_ref[...] = ab
    ge1 = jax.nn.relu(_dot(ab, m1_ref[...]))
    m2_ref[...] = ((1.0 - A) * ge1
                   + A * h1_ref[...].astype(_f32)).astype(_bf)


def _run_cast_l1(adj, m1b, h1b):
    return pl.pallas_call(
        _cast_l1_body, grid=(N // BM_CAST,),
        in_specs=[
            pl.BlockSpec((BM_CAST, N), lambda i: (i, 0)),
            pl.BlockSpec((N, 128), lambda i: (0, 0)),
            pl.BlockSpec((BM_CAST, 128), lambda i: (i, 0)),
        ],
        out_specs=[
            pl.BlockSpec((BM_CAST, N), lambda i: (i, 0)),
            pl.BlockSpec((BM_CAST, 128), lambda i: (i, 0)),
        ],
        out_shape=[
            jax.ShapeDtypeStruct((N, N), _bf),
            jax.ShapeDtypeStruct((N, 128), _bf),
        ])(adj, m1b, h1b)


def _l2_body(adj_ref, m_ref, w_ref, h_ref, out_ref):
    acc = _dot(adj_ref[...], m_ref[...]).astype(_bf)
    ge = jax.nn.relu(_dot(acc, w_ref[...]))
    out_ref[...] = ((1.0 - A) * ge + A * h_ref[...].astype(_f32)).astype(_bf)


def _l3_body(adj_ref, m_ref, w_ref, h_ref, w2_ref, out_ref):
    acc = _dot(adj_ref[...], m_ref[...]).astype(_bf)
    ge = jax.nn.relu(_dot(acc, w_ref[...]))
    mix = ((1.0 - A) * ge + A * h_ref[...].astype(_f32)).astype(_bf)
    out_ref[...] = _dot(mix, w2_ref[...]).astype(_bf)


def _l4_body(adj_ref, m_ref, zae_ref, zgb_ref, m5_ref):
    zg = _dot(adj_ref[...], m_ref[...])
    zgb = zg.astype(_bf)
    zgb_ref[...] = zgb
    zi = ((1.0 - A) * zg + A * zae_ref[...]).astype(_bf)
    m5_ref[...] = jnp.concatenate([zi, zgb], axis=1)


def _l5_body(adj_ref, m_ref, w5_ref, w6_ref, zl_ref, m6_ref):
    acc = _dot(adj_ref[...], m_ref[...])
    zl_ref[...] = acc[:, :64]
    gd1 = jax.nn.relu(_dot(acc[:, 64:].astype(_bf), w5_ref[...])).astype(_bf)
    m6_ref[...] = _dot(gd1, w6_ref[...]).astype(_bf)


def _l6_body(adj_ref, m_ref, w_ref, out_ref):
    gd2 = jax.nn.relu(_dot(adj_ref[...], m_ref[...])).astype(_bf)
    out_ref[...] = _dot(gd2, w_ref[...]).astype(_bf)


def _l7_body(adj_ref, m_ref, zgb_ref, zgfull_ref, out_ref, ah_ref):
    out_ref[...] = jax.nn.relu(_dot(adj_ref[...], m_ref[...])).astype(_bf)
    prod = jax.lax.dot_general(
        zgb_ref[...], zgfull_ref[...], (((1,), (1,)), ((), ())),
        preferred_element_type=_f32)
    ah_ref[...] = jax.nn.sigmoid(prod)


def _softq(z, c, cc, q_ref):
    zz = jnp.sum(z * z, axis=1, keepdims=True)
    zc = jax.lax.dot_general(z, c, (((1,), (1,)), ((), ())),
                             preferred_element_type=_f32,
                             precision=jax.lax.Precision.HIGHEST)
    d2 = zz + cc - 2.0 * zc
    q = 1.0 / (1.0 + d2 / V)
    q = q ** ((V + 1.0) / 2.0)
    q_ref[...] = q / jnp.sum(q, axis=1, keepdims=True)


def _l8_body(adj_ref, m_ref, w_ref, zl_ref, zae_ref, c_ref,
             out_ref, q_ref, q1_ref):
    acc = _dot(adj_ref[...], m_ref[...]).astype(_bf)
    out_ref[...] = jax.nn.relu(_dot(acc, w_ref[...]))
    c = c_ref[...]
    cc = jnp.sum(c * c, axis=1)[None, :]
    _softq(zl_ref[...], c, cc, q_ref)
    _softq(zae_ref[...], c, cc, q1_ref)


def _spmm(body, adjb, m, extras, outs):
    in_specs = [_row(N), _full(m)]
    args = [adjb, m]
    for e, blocked in extras:
        in_specs.append(_row(e.shape[1]) if blocked else _full(e))
        args.append(e)
    out_specs = [_row(c) for c, _ in outs]
    out_shape = [jax.ShapeDtypeStruct((N, c), dt) for c, dt in outs]
    if len(outs) == 1:
        out_specs, out_shape = out_specs[0], out_shape[0]
    return pl.pallas_call(
        body, grid=(N // BM,), in_specs=in_specs, out_specs=out_specs,
        out_shape=out_shape)(*args)


def kernel(x, adj, We1, be1, We2, be2, We3, be3, Wz, bz, Wd1, bd1, Wd2, bd2,
           Wd3, bd3, Wxb, bxb, Wg1, Wg2, Wg3, Wg4, Wg5, Wg6, Wg7, Wg8,
           cluster):
    wb = [w.astype(_bf) for w in (We1, We2, We3, Wz, Wd1, Wd2, Wd3, Wxb)]
    bb = [b.reshape(1, -1) for b in (be1, be2, be3, bz, bd1, bd2, bd3, bxb)]
    wg = [w.astype(_bf) for w in (Wg1, Wg2, Wg3, Wg4, Wg5, Wg6, Wg7, Wg8)]

    h1b, h2b, h3b, z_ae, x_bar, m1b = _run_ae(x, wb, bb, wg[0])

    adjb, m2b = _run_cast_l1(adj, m1b, h1b)
    m3b = _spmm(_l2_body, adjb, m2b, [(wg[1], False), (h2b, True)],
                [(256, _bf)])
    m4b = _spmm(_l3_body, adjb, m3b,
                [(wg[2], False), (h3b, True), (wg[3], False)], [(64, _bf)])
    zgb, m5b = _spmm(_l4_body, adjb, m4b, [(z_ae, True)],
                     [(64, _bf), (128, _bf)])
    z_l, m6b = _spmm(_l5_body, adjb, m5b, [(wg[4], False), (wg[5], False)],
                     [(64, _f32), (256, _bf)])
    m7b = _spmm(_l6_body, adjb, m6b, [(wg[6], False)], [(128, _bf)])
    m8b, adj_hat = _spmm(_l7_body, adjb, m7b, [(zgb, True), (zgb, False)],
                         [(128, _bf), (N, _f32)])
    z_hat, q, q1 = _spmm(
        _l8_body, adjb, m8b,
        [(wg[7], False), (z_l, True), (z_ae, True), (cluster, False)],
        [(512, _f32), (16, _f32), (16, _f32)])
    return (x_bar, z_hat, adj_hat, z_ae, q, q1, z_l)


# speedup vs baseline: 1.5099x; 1.0249x over previous
import jax
import jax.numpy as jnp
from jax.experimental import pallas as pl

N = 10000
A = 0.5
V = 1.0

BM = 400
BM_WIDE = 800
BM_CAST = 400

_bf = jnp.bfloat16
_f32 = jnp.float32


def _dot(a, b):
    return jnp.dot(a, b, preferred_element_type=_f32)


def _row(c):
    return pl.BlockSpec((BM, c), lambda i: (i, 0))


def _full(arr):
    return pl.BlockSpec(arr.shape, lambda i: (0,) * arr.ndim)


def _ae_body(x_ref,
             we1, be1, we2, be2, we3, be3, wz, bz,
             wd1, bd1, wd2, bd2, wd3, bd3, wxb, bxb, wg1,
             h1_ref, h2_ref, h3_ref, zae_ref, xbar_ref, m1_ref):
    xb = x_ref[...].astype(_bf)
    h1 = jax.nn.relu(_dot(xb, we1[...]) + be1[...])
    h1_ref[...] = h1.astype(_bf)
    h2 = jax.nn.relu(_dot(h1.astype(_bf), we2[...]) + be2[...])
    h2_ref[...] = h2.astype(_bf)
    h3 = jax.nn.relu(_dot(h2.astype(_bf), we3[...]) + be3[...])
    h3_ref[...] = h3.astype(_bf)
    zae = _dot(h3.astype(_bf), wz[...]) + bz[...]
    zae_ref[...] = zae
    dd1 = jax.nn.relu(_dot(zae.astype(_bf), wd1[...]) + bd1[...])
    dd2 = jax.nn.relu(_dot(dd1.astype(_bf), wd2[...]) + bd2[...])
    dd3 = jax.nn.relu(_dot(dd2.astype(_bf), wd3[...]) + bd3[...])
    xbar_ref[...] = _dot(dd3.astype(_bf), wxb[...]) + bxb[...]
    m1_ref[...] = _dot(xb, wg1[...]).astype(_bf)


def _run_ae(x, wb, bb, wg1b):
    in_specs = [_row(512)]
    args = [x]
    for w, b in zip(wb, bb):
        in_specs += [_full(w), _full(b)]
        args += [w, b]
    in_specs.append(_full(wg1b))
    args.append(wg1b)
    out_shape = [
        jax.ShapeDtypeStruct((N, 128), _bf),
        jax.ShapeDtypeStruct((N, 256), _bf),
        jax.ShapeDtypeStruct((N, 512), _bf),
        jax.ShapeDtypeStruct((N, 64), _f32),
        jax.ShapeDtypeStruct((N, 512), _f32),
        jax.ShapeDtypeStruct((N, 128), _bf),
    ]
    out_specs = [_row(128), _row(256), _row(512), _row(64), _row(512),
                 _row(128)]
    return pl.pallas_call(
        _ae_body, grid=(N // BM,), in_specs=in_specs, out_specs=out_specs,
        out_shape=out_shape)(*args)


def _cast_l1_body(adj_ref, m1_ref, h1_ref, adjb_ref, m2_ref):
    ab = adj_ref[...].astype(_bf)
    adjb_ref[...] = ab
    ge1 = jax.nn.relu(_dot(ab, m1_ref[...]))
    m2_ref[...] = ((1.0 - A) * ge1
                   + A * h1_ref[...].astype(_f32)).astype(_bf)


def _run_cast_l1(adj, m1b, h1b):
    return pl.pallas_call(
        _cast_l1_body, grid=(N // BM_CAST,),
        in_specs=[
            pl.BlockSpec((BM_CAST, N), lambda i: (i, 0)),
            pl.BlockSpec((N, 128), lambda i: (0, 0)),
            pl.BlockSpec((BM_CAST, 128), lambda i: (i, 0)),
        ],
        out_specs=[
            pl.BlockSpec((BM_CAST, N), lambda i: (i, 0)),
            pl.BlockSpec((BM_CAST, 128), lambda i: (i, 0)),
        ],
        out_shape=[
            jax.ShapeDtypeStruct((N, N), _bf),
            jax.ShapeDtypeStruct((N, 128), _bf),
        ])(adj, m1b, h1b)


def _l2_body(adj_ref, m_ref, w_ref, h_ref, out_ref):
    acc = _dot(adj_ref[...], m_ref[...]).astype(_bf)
    ge = jax.nn.relu(_dot(acc, w_ref[...]))
    out_ref[...] = ((1.0 - A) * ge + A * h_ref[...].astype(_f32)).astype(_bf)


def _l3_body(adj_ref, m_ref, w_ref, h_ref, w2_ref, out_ref):
    acc = _dot(adj_ref[...], m_ref[...]).astype(_bf)
    ge = jax.nn.relu(_dot(acc, w_ref[...]))
    mix = ((1.0 - A) * ge + A * h_ref[...].astype(_f32)).astype(_bf)
    out_ref[...] = _dot(mix, w2_ref[...]).astype(_bf)


def _l4_body(adj_ref, m_ref, zae_ref, zgb_ref, m5_ref):
    zg = _dot(adj_ref[...], m_ref[...])
    zgb = zg.astype(_bf)
    zgb_ref[...] = zgb
    zi = ((1.0 - A) * zg + A * zae_ref[...]).astype(_bf)
    m5_ref[...] = jnp.concatenate([zi, zgb], axis=1)


def _l5_body(adj_ref, m_ref, w5_ref, w6_ref, zl_ref, m6_ref):
    acc = _dot(adj_ref[...], m_ref[...])
    zl_ref[...] = acc[:, :64]
    gd1 = jax.nn.relu(_dot(acc[:, 64:].astype(_bf), w5_ref[...])).astype(_bf)
    m6_ref[...] = _dot(gd1, w6_ref[...]).astype(_bf)


def _l6_body(adj_ref, m_ref, w_ref, out_ref):
    gd2 = jax.nn.relu(_dot(adj_ref[...], m_ref[...])).astype(_bf)
    out_ref[...] = _dot(gd2, w_ref[...]).astype(_bf)


def _l7_body(adj_ref, m_ref, zgb_ref, zgfull_ref, out_ref, ah_ref):
    out_ref[...] = jax.nn.relu(_dot(adj_ref[...], m_ref[...])).astype(_bf)
    prod = jax.lax.dot_general(
        zgb_ref[...], zgfull_ref[...], (((1,), (1,)), ((), ())),
        preferred_element_type=_f32)
    ah_ref[...] = jax.nn.sigmoid(prod)


def _softq(z, c, cc, q_ref):
    zz = jnp.sum(z * z, axis=1, keepdims=True)
    zc = jax.lax.dot_general(z, c, (((1,), (1,)), ((), ())),
                             preferred_element_type=_f32,
                             precision=jax.lax.Precision.HIGHEST)
    d2 = zz + cc - 2.0 * zc
    q = 1.0 / (1.0 + d2 / V)
    q = q ** ((V + 1.0) / 2.0)
    q_ref[...] = q / jnp.sum(q, axis=1, keepdims=True)


def _l8_body(adj_ref, m_ref, w_ref, zl_ref, zae_ref, c_ref,
             out_ref, q_ref, q1_ref):
    acc = _dot(adj_ref[...], m_ref[...]).astype(_bf)
    out_ref[...] = jax.nn.relu(_dot(acc, w_ref[...]))
    c = c_ref[...]
    cc = jnp.sum(c * c, axis=1)[None, :]
    _softq(zl_ref[...], c, cc, q_ref)
    _softq(zae_ref[...], c, cc, q1_ref)


def _spmm(body, adjb, m, extras, outs, bm=BM):
    row = lambda c: pl.BlockSpec((bm, c), lambda i: (i, 0))
    in_specs = [row(N), _full(m)]
    args = [adjb, m]
    for e, blocked in extras:
        in_specs.append(row(e.shape[1]) if blocked else _full(e))
        args.append(e)
    out_specs = [row(c) for c, _ in outs]
    out_shape = [jax.ShapeDtypeStruct((N, c), dt) for c, dt in outs]
    if len(outs) == 1:
        out_specs, out_shape = out_specs[0], out_shape[0]
    return pl.pallas_call(
        body, grid=(pl.cdiv(N, bm),), in_specs=in_specs,
        out_specs=out_specs, out_shape=out_shape)(*args)


def kernel(x, adj, We1, be1, We2, be2, We3, be3, Wz, bz, Wd1, bd1, Wd2, bd2,
           Wd3, bd3, Wxb, bxb, Wg1, Wg2, Wg3, Wg4, Wg5, Wg6, Wg7, Wg8,
           cluster):
    wb = [w.astype(_bf) for w in (We1, We2, We3, Wz, Wd1, Wd2, Wd3, Wxb)]
    bb = [b.reshape(1, -1) for b in (be1, be2, be3, bz, bd1, bd2, bd3, bxb)]
    wg = [w.astype(_bf) for w in (Wg1, Wg2, Wg3, Wg4, Wg5, Wg6, Wg7, Wg8)]

    h1b, h2b, h3b, z_ae, x_bar, m1b = _run_ae(x, wb, bb, wg[0])

    adjb, m2b = _run_cast_l1(adj, m1b, h1b)
    m3b = _spmm(_l2_body, adjb, m2b, [(wg[1], False), (h2b, True)],
                [(256, _bf)], bm=BM_WIDE)
    m4b = _spmm(_l3_body, adjb, m3b,
                [(wg[2], False), (h3b, True), (wg[3], False)], [(64, _bf)],
                bm=BM_WIDE)
    zgb, m5b = _spmm(_l4_body, adjb, m4b, [(z_ae, True)],
                     [(64, _bf), (128, _bf)], bm=BM_WIDE)
    z_l, m6b = _spmm(_l5_body, adjb, m5b, [(wg[4], False), (wg[5], False)],
                     [(64, _f32), (256, _bf)], bm=BM_WIDE)
    m7b = _spmm(_l6_body, adjb, m6b, [(wg[6], False)], [(128, _bf)],
                bm=BM_WIDE)
    m8b, adj_hat = _spmm(_l7_body, adjb, m7b, [(zgb, True), (zgb, False)],
                         [(128, _bf), (N, _f32)])
    z_hat, q, q1 = _spmm(
        _l8_body, adjb, m8b,
        [(wg[7], False), (z_l, True), (z_ae, True), (cluster, False)],
        [(512, _f32), (16, _f32), (16, _f32)], bm=BM_WIDE)
    return (x_bar, z_hat, adj_hat, z_ae, q, q1, z_l)
